# Initial kernel scaffold; baseline (speedup 1.0000x reference)
import jax
import jax.numpy as jnp
from jax.experimental import pallas as pl


def kernel(hidden_states, topk_weights, topk_ids, gate_up_weights, down_weights):
    raise NotImplementedError("write your pallas kernel here")


# speedup vs baseline: 1.2837x; 1.2837x over previous
import functools

import jax
import jax.numpy as jnp
from jax import lax
from jax.experimental import pallas as pl
from jax.experimental.pallas import tpu as pltpu
from jax.experimental.pallas import tpu_sc as plsc

_H = 768
_F = 2048
_E = 8
_K = 2
_T = 2048

_B = 256
_NB = (_K * _T) // _B + _E
_NPAD = _NB * _B

_NC, _NS = 2, 16
_NW = _NC * _NS
_GROWS = _NPAD // _NW
_GCH = _GROWS // 2
_CROWS = _T // _NW

_sc_mesh = plsc.VectorSubcoreMesh(core_axis_name="c", subcore_axis_name="s")


@functools.partial(
    pl.kernel,
    out_type=jax.ShapeDtypeStruct((_NPAD, _H), jnp.float32),
    mesh=_sc_mesh,
    scratch_types=[
        pltpu.VMEM((_GROWS,), jnp.int32),
        pltpu.VMEM((_GCH, _H), jnp.float32),
        pltpu.SemaphoreType.DMA,
    ],
)
def _sc_gather(hidden_hbm, rowtok_hbm, xs_hbm, idx_v, rows_v, sem):
    wid = lax.axis_index("s") * _NC + lax.axis_index("c")
    base = wid * _GROWS
    pltpu.sync_copy(rowtok_hbm.at[pl.ds(base, _GROWS)], idx_v)
    for off in (0, _GCH):
        pltpu.async_copy(
            hidden_hbm.at[idx_v.at[pl.ds(off, _GCH)]], rows_v, sem
        ).wait()
        pltpu.sync_copy(rows_v, xs_hbm.at[pl.ds(base + off, _GCH)])


@functools.partial(
    pl.kernel,
    out_type=jax.ShapeDtypeStruct((_T, _H), jnp.float32),
    mesh=_sc_mesh,
    scratch_types=[
        pltpu.VMEM((_CROWS,), jnp.int32),
        pltpu.VMEM((_CROWS,), jnp.int32),
        pltpu.VMEM((_CROWS, _H), jnp.float32),
        pltpu.VMEM((_CROWS, _H), jnp.float32),
        pltpu.SemaphoreType.DMA,
        pltpu.SemaphoreType.DMA,
    ],
)
def _sc_combine(ys_hbm, i0_hbm, i1_hbm, out_hbm, i0_v, i1_v, a_v, b_v, s0, s1):
    wid = lax.axis_index("s") * _NC + lax.axis_index("c")
    base = wid * _CROWS
    pltpu.sync_copy(i0_hbm.at[pl.ds(base, _CROWS)], i0_v)
    pltpu.sync_copy(i1_hbm.at[pl.ds(base, _CROWS)], i1_v)
    ca = pltpu.async_copy(ys_hbm.at[i0_v], a_v, s0)
    cb = pltpu.async_copy(ys_hbm.at[i1_v], b_v, s1)
    ca.wait()
    cb.wait()

    def _row(i, carry):
        for k in range(_H // 16):
            sl = pl.ds(k * 16, 16)
            a_v[i, sl] = a_v[i, sl] + b_v[i, sl]
        return carry

    lax.fori_loop(0, _CROWS, _row, 0)
    pltpu.sync_copy(a_v, out_hbm.at[pl.ds(base, _CROWS)])


def _tc_body(be_ref, xs_ref, rw_ref, gu_ref, dn_ref, ys_ref):
    x = xs_ref[...]
    z = lax.dot_general(
        x, gu_ref[0], (((1,), (1,)), ((), ())),
        preferred_element_type=jnp.float32,
    )
    gate = z[:, :_F]
    up = z[:, _F:]
    act = gate * jax.nn.sigmoid(gate) * up
    y = lax.dot_general(
        act, dn_ref[0], (((1,), (1,)), ((), ())),
        preferred_element_type=jnp.float32,
    )
    ys_ref[...] = y * rw_ref[...]


_tc_grid = pltpu.PrefetchScalarGridSpec(
    num_scalar_prefetch=1,
    grid=(_NB,),
    in_specs=[
        pl.BlockSpec((_B, _H), lambda b, be: (b, 0)),
        pl.BlockSpec((_B, 1), lambda b, be: (b, 0)),
        pl.BlockSpec((1, 2 * _F, _H), lambda b, be: (be[b], 0, 0)),
        pl.BlockSpec((1, _H, _F), lambda b, be: (be[b], 0, 0)),
    ],
    out_specs=pl.BlockSpec((_B, _H), lambda b, be: (b, 0)),
)

_tc_moe = pl.pallas_call(
    _tc_body,
    grid_spec=_tc_grid,
    out_shape=jax.ShapeDtypeStruct((_NPAD, _H), jnp.float32),
)


def kernel(hidden_states, topk_weights, topk_ids, gate_up_weights, down_weights):
    ids = topk_ids.astype(jnp.int32)
    flat_e = ids.reshape(-1)
    onehot = (flat_e[:, None] == jnp.arange(_E, dtype=jnp.int32)[None, :]).astype(jnp.int32)
    prefix = jnp.cumsum(onehot, axis=0)
    counts = prefix[-1]
    rank = jnp.take_along_axis(prefix, flat_e[:, None], axis=1)[:, 0] - 1
    padded = ((counts + _B - 1) // _B) * _B
    ends = jnp.cumsum(padded)
    seg_start = ends - padded
    dest = seg_start[flat_e] + rank
    inv = dest.reshape(_T, _K)
    row_tok = jnp.zeros((_NPAD,), jnp.int32).at[dest].set(
        jnp.arange(_K * _T, dtype=jnp.int32) // _K)
    row_w = jnp.zeros((_NPAD,), jnp.float32).at[dest].set(
        topk_weights.reshape(-1))
    total = ends[-1]
    blk_start = jnp.arange(_NB, dtype=jnp.int32) * _B
    be = jnp.searchsorted(ends, blk_start, side="right").astype(jnp.int32)
    last_e = jnp.max(jnp.where(counts > 0, jnp.arange(_E, dtype=jnp.int32), 0))
    be = jnp.where(blk_start < total, jnp.minimum(be, _E - 1), last_e)

    xs = _sc_gather(hidden_states, row_tok)
    ys = _tc_moe(be, xs, row_w.reshape(_NPAD, 1),
                 gate_up_weights, down_weights)
    out = _sc_combine(ys, inv[:, 0].astype(jnp.int32),
                      inv[:, 1].astype(jnp.int32))
    return out


# speedup vs baseline: 1.6305x; 1.2702x over previous
import functools

import jax
import jax.numpy as jnp
from jax import lax
from jax.experimental import pallas as pl
from jax.experimental.pallas import tpu as pltpu
from jax.experimental.pallas import tpu_sc as plsc

_H = 768
_F = 2048
_E = 8
_K = 2
_T = 2048

_B = 256
_NB = (_K * _T) // _B + _E
_NPAD = _NB * _B

_NC, _NS = 2, 16
_NW = _NC * _NS
_CROWS = _T // _NW

_sc_mesh = plsc.VectorSubcoreMesh(core_axis_name="c", subcore_axis_name="s")


@functools.partial(
    pl.kernel,
    out_type=jax.ShapeDtypeStruct((_T, _H), jnp.float32),
    mesh=_sc_mesh,
    scratch_types=[
        pltpu.VMEM((_CROWS,), jnp.int32),
        pltpu.VMEM((_CROWS,), jnp.int32),
        pltpu.VMEM((_CROWS, _H), jnp.float32),
        pltpu.VMEM((_CROWS, _H), jnp.float32),
        pltpu.SemaphoreType.DMA,
        pltpu.SemaphoreType.DMA,
    ],
)
def _sc_combine(ys_hbm, i0_hbm, i1_hbm, out_hbm, i0_v, i1_v, a_v, b_v, s0, s1):
    wid = lax.axis_index("s") * _NC + lax.axis_index("c")
    base = wid * _CROWS
    pltpu.sync_copy(i0_hbm.at[pl.ds(base, _CROWS)], i0_v)
    pltpu.sync_copy(i1_hbm.at[pl.ds(base, _CROWS)], i1_v)
    ca = pltpu.async_copy(ys_hbm.at[i0_v], a_v, s0)
    cb = pltpu.async_copy(ys_hbm.at[i1_v], b_v, s1)
    ca.wait()
    cb.wait()

    def _row(i, carry):
        for k in range(_H // 16):
            sl = pl.ds(k * 16, 16)
            a_v[i, sl] = a_v[i, sl] + b_v[i, sl]
        return carry

    lax.fori_loop(0, _CROWS, _row, 0)
    pltpu.sync_copy(a_v, out_hbm.at[pl.ds(base, _CROWS)])


def _tc_body(be_ref, tok_ref, rw_ref, hid_ref, gu_ref, dn_ref, ys_ref):
    ids = tok_ref[...]
    onehot = jnp.where(
        ids == lax.broadcasted_iota(jnp.int32, (_B, _T), 1),
        1.0, 0.0).astype(jnp.float32)
    x = lax.dot_general(
        onehot, hid_ref[...], (((1,), (0,)), ((), ())),
        preferred_element_type=jnp.float32,
    )
    z = lax.dot_general(
        x, gu_ref[0], (((1,), (1,)), ((), ())),
        preferred_element_type=jnp.float32,
    )
    gate = z[:, :_F]
    up = z[:, _F:]
    act = gate * jax.nn.sigmoid(gate) * up
    y = lax.dot_general(
        act, dn_ref[0], (((1,), (1,)), ((), ())),
        preferred_element_type=jnp.float32,
    )
    ys_ref[...] = y * rw_ref[...]


_tc_grid = pltpu.PrefetchScalarGridSpec(
    num_scalar_prefetch=1,
    grid=(_NB,),
    in_specs=[
        pl.BlockSpec((_B, 1), lambda b, be: (b, 0)),
        pl.BlockSpec((_B, 1), lambda b, be: (b, 0)),
        pl.BlockSpec((_T, _H), lambda b, be: (0, 0)),
        pl.BlockSpec((1, 2 * _F, _H), lambda b, be: (be[b], 0, 0)),
        pl.BlockSpec((1, _H, _F), lambda b, be: (be[b], 0, 0)),
    ],
    out_specs=pl.BlockSpec((_B, _H), lambda b, be: (b, 0)),
)

_tc_moe = pl.pallas_call(
    _tc_body,
    grid_spec=_tc_grid,
    out_shape=jax.ShapeDtypeStruct((_NPAD, _H), jnp.float32),
)


def kernel(hidden_states, topk_weights, topk_ids, gate_up_weights, down_weights):
    ids = topk_ids.astype(jnp.int32)
    flat_e = ids.reshape(-1)
    onehot = (flat_e[:, None] == jnp.arange(_E, dtype=jnp.int32)[None, :]).astype(jnp.int32)
    prefix = jnp.cumsum(onehot, axis=0)
    counts = prefix[-1]
    rank = jnp.take_along_axis(prefix, flat_e[:, None], axis=1)[:, 0] - 1
    padded = ((counts + _B - 1) // _B) * _B
    ends = jnp.cumsum(padded)
    seg_start = ends - padded
    dest = seg_start[flat_e] + rank
    inv = dest.reshape(_T, _K)
    row_tok = jnp.zeros((_NPAD,), jnp.int32).at[dest].set(
        jnp.arange(_K * _T, dtype=jnp.int32) // _K)
    row_w = jnp.zeros((_NPAD,), jnp.float32).at[dest].set(
        topk_weights.reshape(-1))
    total = ends[-1]
    blk_start = jnp.arange(_NB, dtype=jnp.int32) * _B
    be = jnp.searchsorted(ends, blk_start, side="right").astype(jnp.int32)
    last_e = jnp.max(jnp.where(counts > 0, jnp.arange(_E, dtype=jnp.int32), 0))
    be = jnp.where(blk_start < total, jnp.minimum(be, _E - 1), last_e)

    ys = _tc_moe(be, row_tok.reshape(_NPAD, 1), row_w.reshape(_NPAD, 1),
                 hidden_states, gate_up_weights, down_weights)
    out = _sc_combine(ys, inv[:, 0].astype(jnp.int32),
                      inv[:, 1].astype(jnp.int32))
    return out


# speedup vs baseline: 1.9741x; 1.2107x over previous
import functools

import jax
import jax.numpy as jnp
from jax import lax
from jax.experimental import pallas as pl
from jax.experimental.pallas import tpu as pltpu
from jax.experimental.pallas import tpu_sc as plsc

_H = 768
_F = 2048
_E = 8
_K = 2
_T = 2048

_B = 256
_NB = (_K * _T) // _B + _E - 1
_NPAD = _NB * _B

_NC, _NS = 2, 16
_NW = _NC * _NS
_CROWS = _T // _NW

_sc_mesh = plsc.VectorSubcoreMesh(core_axis_name="c", subcore_axis_name="s")


@functools.partial(
    pl.kernel,
    out_type=jax.ShapeDtypeStruct((_T, _H), jnp.float32),
    mesh=_sc_mesh,
    scratch_types=[
        pltpu.VMEM((_CROWS,), jnp.int32),
        pltpu.VMEM((_CROWS,), jnp.int32),
        pltpu.VMEM((_CROWS, _H), jnp.float32),
        pltpu.VMEM((_CROWS, _H), jnp.float32),
        pltpu.SemaphoreType.DMA,
        pltpu.SemaphoreType.DMA,
    ],
)
def _sc_combine(ys_hbm, i0_hbm, i1_hbm, out_hbm, i0_v, i1_v, a_v, b_v, s0, s1):
    wid = lax.axis_index("s") * _NC + lax.axis_index("c")
    base = wid * _CROWS
    pltpu.sync_copy(i0_hbm.at[pl.ds(base, _CROWS)], i0_v)
    pltpu.sync_copy(i1_hbm.at[pl.ds(base, _CROWS)], i1_v)
    ca = pltpu.async_copy(ys_hbm.at[i0_v], a_v, s0)
    cb = pltpu.async_copy(ys_hbm.at[i1_v], b_v, s1)
    ca.wait()
    cb.wait()

    def _row(i, carry):
        for k in range(_H // 16):
            sl = pl.ds(k * 16, 16)
            a_v[i, sl] = a_v[i, sl] + b_v[i, sl]
        return carry

    lax.fori_loop(0, _CROWS, _row, 0)
    pltpu.sync_copy(a_v, out_hbm.at[pl.ds(base, _CROWS)])


def _tc_body(be_ref, i0_ref, i1_ref, w0_ref, w1_ref, hid_ref, gu_ref, dn_ref,
             ys_ref):
    row_id = (pl.program_id(0) * _B
              + lax.broadcasted_iota(jnp.int32, (_B, _T), 0))
    hit0 = i0_ref[...] == row_id
    hit1 = i1_ref[...] == row_id
    onehot = jnp.where(hit0 | hit1, 1.0, 0.0).astype(jnp.float32)
    zero = jnp.zeros((), jnp.float32)
    rw = jnp.sum(jnp.where(hit0, w0_ref[...], zero)
                 + jnp.where(hit1, w1_ref[...], zero),
                 axis=1, keepdims=True)
    x = lax.dot_general(
        onehot, hid_ref[...], (((1,), (0,)), ((), ())),
        preferred_element_type=jnp.float32,
    )
    z = lax.dot_general(
        x, gu_ref[0], (((1,), (1,)), ((), ())),
        preferred_element_type=jnp.float32,
    )
    gate = z[:, :_F]
    up = z[:, _F:]
    act = gate * jax.nn.sigmoid(gate) * up
    y = lax.dot_general(
        act, dn_ref[0], (((1,), (1,)), ((), ())),
        preferred_element_type=jnp.float32,
    )
    ys_ref[...] = y * rw


_tc_grid = pltpu.PrefetchScalarGridSpec(
    num_scalar_prefetch=1,
    grid=(_NB,),
    in_specs=[
        pl.BlockSpec((1, _T), lambda b, be: (0, 0)),
        pl.BlockSpec((1, _T), lambda b, be: (0, 0)),
        pl.BlockSpec((1, _T), lambda b, be: (0, 0)),
        pl.BlockSpec((1, _T), lambda b, be: (0, 0)),
        pl.BlockSpec((_T, _H), lambda b, be: (0, 0)),
        pl.BlockSpec((1, 2 * _F, _H), lambda b, be: (be[b], 0, 0)),
        pl.BlockSpec((1, _H, _F), lambda b, be: (be[b], 0, 0)),
    ],
    out_specs=pl.BlockSpec((_B, _H), lambda b, be: (b, 0)),
)

_tc_moe = pl.pallas_call(
    _tc_body,
    grid_spec=_tc_grid,
    out_shape=jax.ShapeDtypeStruct((_NPAD, _H), jnp.float32),
)


def kernel(hidden_states, topk_weights, topk_ids, gate_up_weights, down_weights):
    ids = topk_ids.astype(jnp.int32)
    flat_e = ids.reshape(-1)
    onehot = (flat_e[:, None] == jnp.arange(_E, dtype=jnp.int32)[None, :]).astype(jnp.int32)
    prefix = jnp.cumsum(onehot, axis=0)
    counts = prefix[-1]
    rank = jnp.take_along_axis(prefix, flat_e[:, None], axis=1)[:, 0] - 1
    padded = ((counts + _B - 1) // _B) * _B
    ends = jnp.cumsum(padded)
    seg_start = ends - padded
    dest = seg_start[flat_e] + rank
    inv = dest.reshape(_T, _K)
    total = ends[-1]
    blk_start = jnp.arange(_NB, dtype=jnp.int32) * _B
    be = jnp.searchsorted(ends, blk_start, side="right").astype(jnp.int32)
    last_e = jnp.max(jnp.where(counts > 0, jnp.arange(_E, dtype=jnp.int32), 0))
    be = jnp.where(blk_start < total, jnp.minimum(be, _E - 1), last_e)

    i0 = inv[:, 0].astype(jnp.int32)
    i1 = inv[:, 1].astype(jnp.int32)
    ys = _tc_moe(be, i0.reshape(1, _T), i1.reshape(1, _T),
                 topk_weights[:, 0].reshape(1, _T),
                 topk_weights[:, 1].reshape(1, _T),
                 hidden_states, gate_up_weights, down_weights)
    out = _sc_combine(ys, i0, i1)
    return out


# speedup vs baseline: 2.1029x; 1.0652x over previous
import functools

import jax
import jax.numpy as jnp
from jax import lax
from jax.experimental import pallas as pl
from jax.experimental.pallas import tpu as pltpu
from jax.experimental.pallas import tpu_sc as plsc

_H = 768
_F = 2048
_E = 8
_K = 2
_T = 2048

_B = 256
_NB = (_K * _T) // _B + _E - 1
_NPAD = _NB * _B

_NC, _NS = 2, 16
_NW = _NC * _NS
_CROWS = _T // _NW

_sc_mesh = plsc.VectorSubcoreMesh(core_axis_name="c", subcore_axis_name="s")


@functools.partial(
    pl.kernel,
    out_type=jax.ShapeDtypeStruct((_T, _H), jnp.float32),
    mesh=_sc_mesh,
    scratch_types=[
        pltpu.VMEM((_CROWS,), jnp.int32),
        pltpu.VMEM((_CROWS,), jnp.int32),
        pltpu.VMEM((_CROWS, _H), jnp.float32),
        pltpu.VMEM((_CROWS, _H), jnp.float32),
        pltpu.SemaphoreType.DMA,
        pltpu.SemaphoreType.DMA,
    ],
)
def _sc_combine(ys_hbm, i0_hbm, i1_hbm, out_hbm, i0_v, i1_v, a_v, b_v, s0, s1):
    wid = lax.axis_index("s") * _NC + lax.axis_index("c")
    base = wid * _CROWS
    pltpu.sync_copy(i0_hbm.at[pl.ds(base, _CROWS)], i0_v)
    pltpu.sync_copy(i1_hbm.at[pl.ds(base, _CROWS)], i1_v)
    ca = pltpu.async_copy(ys_hbm.at[i0_v], a_v, s0)
    cb = pltpu.async_copy(ys_hbm.at[i1_v], b_v, s1)
    ca.wait()
    cb.wait()

    def _row(i, carry):
        for k in range(_H // 16):
            sl = pl.ds(k * 16, 16)
            a_v[i, sl] = a_v[i, sl] + b_v[i, sl]
        return carry

    lax.fori_loop(0, _CROWS, _row, 0)
    pltpu.sync_copy(a_v, out_hbm.at[pl.ds(base, _CROWS)])


def _wcopy(gu_hbm, dn_hbm, gu_v, dn_v, sg, sd, e, slot):
    cg = pltpu.make_async_copy(gu_hbm.at[e], gu_v.at[slot], sg.at[slot])
    cd = pltpu.make_async_copy(dn_hbm.at[e], dn_v.at[slot], sd.at[slot])
    return cg, cd


def _tc_body(be_ref, rf_ref, sl_ref, pf_ref, i0_ref, i1_ref, w0_ref, w1_ref,
             hid_ref, gu_hbm, dn_hbm, ys_ref, gu_v, dn_v, sg, sd):
    b = pl.program_id(0)
    slot = sl_ref[b]

    @pl.when(b == 0)
    def _prologue():
        cg, cd = _wcopy(gu_hbm, dn_hbm, gu_v, dn_v, sg, sd, be_ref[0], 0)
        cg.start()
        cd.start()

    @pl.when(pf_ref[b] >= 0)
    def _prefetch_next_run():
        cg, cd = _wcopy(gu_hbm, dn_hbm, gu_v, dn_v, sg, sd,
                        pf_ref[b], 1 - slot)
        cg.start()
        cd.start()

    @pl.when(rf_ref[b] == 1)
    def _wait_this_run():
        cg, cd = _wcopy(gu_hbm, dn_hbm, gu_v, dn_v, sg, sd, be_ref[b], slot)
        cg.wait()
        cd.wait()

    row_id = b * _B + lax.broadcasted_iota(jnp.int32, (_B, _T), 0)
    hit0 = i0_ref[...] == row_id
    hit1 = i1_ref[...] == row_id
    onehot = jnp.where(hit0 | hit1, 1.0, 0.0).astype(jnp.float32)
    zero = jnp.zeros((), jnp.float32)
    rw = jnp.sum(jnp.where(hit0, w0_ref[...], zero)
                 + jnp.where(hit1, w1_ref[...], zero),
                 axis=1, keepdims=True)
    x = lax.dot_general(
        onehot, hid_ref[...], (((1,), (0,)), ((), ())),
        preferred_element_type=jnp.float32,
    )
    z = lax.dot_general(
        x, gu_v[slot], (((1,), (1,)), ((), ())),
        preferred_element_type=jnp.float32,
    )
    gate = z[:, :_F]
    up = z[:, _F:]
    act = gate * jax.nn.sigmoid(gate) * up
    y = lax.dot_general(
        act, dn_v[slot], (((1,), (1,)), ((), ())),
        preferred_element_type=jnp.float32,
    )
    ys_ref[...] = y * rw


_tc_grid = pltpu.PrefetchScalarGridSpec(
    num_scalar_prefetch=4,
    grid=(_NB,),
    in_specs=[
        pl.BlockSpec((1, _T), lambda b, *_: (0, 0)),
        pl.BlockSpec((1, _T), lambda b, *_: (0, 0)),
        pl.BlockSpec((1, _T), lambda b, *_: (0, 0)),
        pl.BlockSpec((1, _T), lambda b, *_: (0, 0)),
        pl.BlockSpec((_T, _H), lambda b, *_: (0, 0)),
        pl.BlockSpec(memory_space=pl.ANY),
        pl.BlockSpec(memory_space=pl.ANY),
    ],
    out_specs=pl.BlockSpec((_B, _H), lambda b, *_: (b, 0)),
    scratch_shapes=[
        pltpu.VMEM((2, 2 * _F, _H), jnp.float32),
        pltpu.VMEM((2, _H, _F), jnp.float32),
        pltpu.SemaphoreType.DMA((2,)),
        pltpu.SemaphoreType.DMA((2,)),
    ],
)

_tc_moe = pl.pallas_call(
    _tc_body,
    grid_spec=_tc_grid,
    out_shape=jax.ShapeDtypeStruct((_NPAD, _H), jnp.float32),
)


def kernel(hidden_states, topk_weights, topk_ids, gate_up_weights, down_weights):
    ids = topk_ids.astype(jnp.int32)
    flat_e = ids.reshape(-1)
    onehot = (flat_e[:, None] == jnp.arange(_E, dtype=jnp.int32)[None, :]).astype(jnp.int32)
    prefix = jnp.cumsum(onehot, axis=0)
    counts = prefix[-1]
    rank = jnp.take_along_axis(prefix, flat_e[:, None], axis=1)[:, 0] - 1
    padded = ((counts + _B - 1) // _B) * _B
    ends = jnp.cumsum(padded)
    seg_start = ends - padded
    dest = seg_start[flat_e] + rank
    inv = dest.reshape(_T, _K)
    total = ends[-1]
    blk_start = jnp.arange(_NB, dtype=jnp.int32) * _B
    be = jnp.searchsorted(ends, blk_start, side="right").astype(jnp.int32)
    last_e = jnp.max(jnp.where(counts > 0, jnp.arange(_E, dtype=jnp.int32), 0))
    be = jnp.where(blk_start < total, jnp.minimum(be, _E - 1), last_e)

    rf = jnp.concatenate([jnp.ones((1,), jnp.int32),
                          (be[1:] != be[:-1]).astype(jnp.int32)])
    run_id = jnp.cumsum(rf) - 1
    slot_arr = (run_id % 2).astype(jnp.int32)
    cand = jnp.where(rf == 1, jnp.arange(_NB, dtype=jnp.int32), _NB)
    sufmin = lax.cummin(cand[::-1])[::-1]
    nxt = jnp.concatenate([sufmin[1:], jnp.full((1,), _NB, jnp.int32)])
    pf = jnp.where((rf == 1) & (nxt < _NB),
                   be[jnp.minimum(nxt, _NB - 1)], -1).astype(jnp.int32)

    i0 = inv[:, 0].astype(jnp.int32)
    i1 = inv[:, 1].astype(jnp.int32)
    ys = _tc_moe(be, rf, slot_arr, pf, i0.reshape(1, _T), i1.reshape(1, _T),
                 topk_weights[:, 0].reshape(1, _T),
                 topk_weights[:, 1].reshape(1, _T),
                 hidden_states, gate_up_weights, down_weights)
    out = _sc_combine(ys, i0, i1)
    return out


# speedup vs baseline: 2.2993x; 1.0934x over previous
import functools

import jax
import jax.numpy as jnp
from jax import lax
from jax.experimental import pallas as pl
from jax.experimental.pallas import tpu as pltpu
from jax.experimental.pallas import tpu_sc as plsc

_H = 768
_F = 2048
_E = 8
_K = 2
_T = 2048

_B = 256
_NB = (_K * _T) // _B + _E - 1
_NPAD = _NB * _B

_NC, _NS = 2, 16
_NW = _NC * _NS
_CROWS = _T // _NW

_sc_mesh = plsc.VectorSubcoreMesh(core_axis_name="c", subcore_axis_name="s")


@functools.partial(
    pl.kernel,
    out_type=jax.ShapeDtypeStruct((_T, _H), jnp.float32),
    mesh=_sc_mesh,
    scratch_types=[
        pltpu.VMEM((_CROWS,), jnp.int32),
        pltpu.VMEM((_CROWS,), jnp.int32),
        pltpu.VMEM((_CROWS, _H), jnp.float32),
        pltpu.VMEM((_CROWS, _H), jnp.float32),
        pltpu.SemaphoreType.DMA,
        pltpu.SemaphoreType.DMA,
    ],
)
def _sc_combine(ys_hbm, i0_hbm, i1_hbm, out_hbm, i0_v, i1_v, a_v, b_v, s0, s1):
    wid = lax.axis_index("s") * _NC + lax.axis_index("c")
    base = wid * _CROWS
    pltpu.sync_copy(i0_hbm.at[pl.ds(base, _CROWS)], i0_v)
    pltpu.sync_copy(i1_hbm.at[pl.ds(base, _CROWS)], i1_v)
    ca = pltpu.async_copy(ys_hbm.at[i0_v], a_v, s0)
    cb = pltpu.async_copy(ys_hbm.at[i1_v], b_v, s1)
    ca.wait()
    cb.wait()

    def _row(i, carry):
        for k in range(_H // 16):
            sl = pl.ds(k * 16, 16)
            a_v[i, sl] = a_v[i, sl] + b_v[i, sl]
        return carry

    lax.fori_loop(0, _CROWS, _row, 0)
    pltpu.sync_copy(a_v, out_hbm.at[pl.ds(base, _CROWS)])


def _wcopy(gu_hbm, dn_hbm, gu_v, dn_v, sg, sd, e, slot):
    cg = pltpu.make_async_copy(gu_hbm.at[e], gu_v.at[slot], sg.at[slot])
    cd = pltpu.make_async_copy(dn_hbm.at[e], dn_v.at[slot], sd.at[slot])
    return cg, cd


def _tc_body(be_ref, rf_ref, sl_ref, pf_ref, rl_ref, i0_ref, i1_ref, w0_ref,
             w1_ref, hid_ref, gu_hbm, dn_hbm, ys_ref, gu_v, dn_v, sg, sd):
    b = pl.program_id(0)
    slot = sl_ref[b]

    @pl.when(b == 0)
    def _prologue():
        cg, cd = _wcopy(gu_hbm, dn_hbm, gu_v, dn_v, sg, sd, be_ref[0], 0)
        cg.start()
        cd.start()

    @pl.when((pf_ref[b] >= 0) & (b > 0))
    def _prefetch_next_run():
        cg, cd = _wcopy(gu_hbm, dn_hbm, gu_v, dn_v, sg, sd,
                        pf_ref[b], 1 - slot)
        cg.start()
        cd.start()

    @pl.when(rf_ref[b] == 1)
    def _wait_this_run():
        cg, cd = _wcopy(gu_hbm, dn_hbm, gu_v, dn_v, sg, sd, be_ref[b], slot)
        cg.wait()
        cd.wait()

    @pl.when((pf_ref[b] >= 0) & (b == 0))
    def _prefetch_after_prologue():
        cg, cd = _wcopy(gu_hbm, dn_hbm, gu_v, dn_v, sg, sd,
                        pf_ref[b], 1 - slot)
        cg.start()
        cd.start()

    @pl.when(rl_ref[b] == 1)
    def _compute():
        row_id = b * _B + lax.broadcasted_iota(jnp.int32, (_B, _T), 0)
        hit0 = i0_ref[...] == row_id
        hit1 = i1_ref[...] == row_id
        onehot = jnp.where(hit0 | hit1, 1.0, 0.0).astype(jnp.float32)
        zero = jnp.zeros((), jnp.float32)
        rw = jnp.sum(jnp.where(hit0, w0_ref[...], zero)
                     + jnp.where(hit1, w1_ref[...], zero),
                     axis=1, keepdims=True)
        x = lax.dot_general(
            onehot, hid_ref[...], (((1,), (0,)), ((), ())),
            preferred_element_type=jnp.float32,
        )
        z = lax.dot_general(
            x, gu_v[slot], (((1,), (1,)), ((), ())),
            preferred_element_type=jnp.float32,
        )
        gate = z[:, :_F]
        up = z[:, _F:]
        act = gate * jax.nn.sigmoid(gate) * up
        y = lax.dot_general(
            act, dn_v[slot], (((1,), (1,)), ((), ())),
            preferred_element_type=jnp.float32,
        )
        ys_ref[...] = y * rw


_tc_grid = pltpu.PrefetchScalarGridSpec(
    num_scalar_prefetch=5,
    grid=(_NB,),
    in_specs=[
        pl.BlockSpec((1, _T), lambda b, *_: (0, 0)),
        pl.BlockSpec((1, _T), lambda b, *_: (0, 0)),
        pl.BlockSpec((1, _T), lambda b, *_: (0, 0)),
        pl.BlockSpec((1, _T), lambda b, *_: (0, 0)),
        pl.BlockSpec((_T, _H), lambda b, *_: (0, 0)),
        pl.BlockSpec(memory_space=pl.ANY),
        pl.BlockSpec(memory_space=pl.ANY),
    ],
    out_specs=pl.BlockSpec((_B, _H), lambda b, *_: (b, 0)),
    scratch_shapes=[
        pltpu.VMEM((2, 2 * _F, _H), jnp.float32),
        pltpu.VMEM((2, _H, _F), jnp.float32),
        pltpu.SemaphoreType.DMA((2,)),
        pltpu.SemaphoreType.DMA((2,)),
    ],
)

_tc_moe = pl.pallas_call(
    _tc_body,
    grid_spec=_tc_grid,
    out_shape=jax.ShapeDtypeStruct((_NPAD, _H), jnp.float32),
)


def kernel(hidden_states, topk_weights, topk_ids, gate_up_weights, down_weights):
    ids = topk_ids.astype(jnp.int32)
    flat_e = ids.reshape(-1)
    onehot = (flat_e[:, None] == jnp.arange(_E, dtype=jnp.int32)[None, :]).astype(jnp.int32)
    prefix = jnp.cumsum(onehot, axis=0)
    counts = prefix[-1]
    rank = jnp.take_along_axis(prefix, flat_e[:, None], axis=1)[:, 0] - 1
    padded = ((counts + _B - 1) // _B) * _B
    ends = jnp.cumsum(padded)
    seg_start = ends - padded
    dest = seg_start[flat_e] + rank
    inv = dest.reshape(_T, _K)
    total = ends[-1]
    blk_start = jnp.arange(_NB, dtype=jnp.int32) * _B
    be = jnp.searchsorted(ends, blk_start, side="right").astype(jnp.int32)
    last_e = jnp.max(jnp.where(counts > 0, jnp.arange(_E, dtype=jnp.int32), 0))
    be = jnp.where(blk_start < total, jnp.minimum(be, _E - 1), last_e)

    rf = jnp.concatenate([jnp.ones((1,), jnp.int32),
                          (be[1:] != be[:-1]).astype(jnp.int32)])
    run_id = jnp.cumsum(rf) - 1
    slot_arr = (run_id % 2).astype(jnp.int32)
    cand = jnp.where(rf == 1, jnp.arange(_NB, dtype=jnp.int32), _NB)
    sufmin = lax.cummin(cand[::-1])[::-1]
    nxt = jnp.concatenate([sufmin[1:], jnp.full((1,), _NB, jnp.int32)])
    pf = jnp.where((rf == 1) & (nxt < _NB),
                   be[jnp.minimum(nxt, _NB - 1)], -1).astype(jnp.int32)
    real = (blk_start < total).astype(jnp.int32)

    i0 = inv[:, 0].astype(jnp.int32)
    i1 = inv[:, 1].astype(jnp.int32)
    ys = _tc_moe(be, rf, slot_arr, pf, real,
                 i0.reshape(1, _T), i1.reshape(1, _T),
                 topk_weights[:, 0].reshape(1, _T),
                 topk_weights[:, 1].reshape(1, _T),
                 hidden_states, gate_up_weights, down_weights)
    out = _sc_combine(ys, i0, i1)
    return out


# speedup vs baseline: 2.4702x; 1.0743x over previous
import functools

import jax
import jax.numpy as jnp
from jax import lax
from jax.experimental import pallas as pl
from jax.experimental.pallas import tpu as pltpu
from jax.experimental.pallas import tpu_sc as plsc

_H = 768
_F = 2048
_E = 8
_K = 2
_T = 2048

_B = 256
_NB = (_K * _T) // _B + _E - 1
_NPAD = _NB * _B

_NC, _NS = 2, 16
_NW = _NC * _NS
_CROWS = _T // _NW

_sc_mesh = plsc.VectorSubcoreMesh(core_axis_name="c", subcore_axis_name="s")


@functools.partial(
    pl.kernel,
    out_type=jax.ShapeDtypeStruct((_T, _H), jnp.float32),
    mesh=_sc_mesh,
    scratch_types=[
        pltpu.VMEM((_CROWS,), jnp.int32),
        pltpu.VMEM((_CROWS,), jnp.int32),
        pltpu.VMEM((_CROWS, _H), jnp.float32),
        pltpu.SemaphoreType.DMA,
        pltpu.SemaphoreType.DMA,
    ],
)
def _sc_combine(ys_hbm, i0_hbm, i1_hbm, out_hbm, i0_v, i1_v, a_v, s0, s1):
    wid = lax.axis_index("s") * _NC + lax.axis_index("c")
    base = wid * _CROWS
    pltpu.sync_copy(i0_hbm.at[pl.ds(base, _CROWS)], i0_v)
    pltpu.sync_copy(i1_hbm.at[pl.ds(base, _CROWS)], i1_v)
    pltpu.async_copy(ys_hbm.at[i0_v], a_v, s0).wait()
    pltpu.async_copy(ys_hbm.at[i1_v], a_v, s1, add=True).wait()
    pltpu.sync_copy(a_v, out_hbm.at[pl.ds(base, _CROWS)])


def _gu_copy(gu_hbm, gu_v, sg, e, slot):
    return pltpu.make_async_copy(gu_hbm.at[e], gu_v.at[slot], sg.at[slot])


def _dn_copy(dn_hbm, dn_v, sd, e, slot):
    return pltpu.make_async_copy(dn_hbm.at[e], dn_v.at[slot], sd.at[slot])


def _tc_body(be_ref, rf_ref, sl_ref, pf_ref, rl_ref, i0_ref, i1_ref, w0_ref,
             w1_ref, hid_ref, gu_hbm, dn_hbm, ys_ref, gu_v, dn_v, sg, sd):
    b = pl.program_id(0)
    slot = sl_ref[b]

    @pl.when(b == 0)
    def _prologue():
        _gu_copy(gu_hbm, gu_v, sg, be_ref[0], 0).start()
        _dn_copy(dn_hbm, dn_v, sd, be_ref[0], 0).start()

    @pl.when((pf_ref[b] >= 0) & (b > 0))
    def _prefetch_next_run():
        _gu_copy(gu_hbm, gu_v, sg, pf_ref[b], 1 - slot).start()
        _dn_copy(dn_hbm, dn_v, sd, pf_ref[b], 1 - slot).start()

    @pl.when(rf_ref[b] == 1)
    def _wait_gu():
        _gu_copy(gu_hbm, gu_v, sg, be_ref[b], slot).wait()

    @pl.when((pf_ref[b] >= 0) & (b == 0))
    def _prefetch_after_prologue():
        _gu_copy(gu_hbm, gu_v, sg, pf_ref[b], 1 - slot).start()
        _dn_copy(dn_hbm, dn_v, sd, pf_ref[b], 1 - slot).start()

    @pl.when(rl_ref[b] == 1)
    def _compute():
        row_id = b * _B + lax.broadcasted_iota(jnp.int32, (_B, _T), 0)
        hit0 = i0_ref[...] == row_id
        hit1 = i1_ref[...] == row_id
        onehot = jnp.where(hit0 | hit1, 1.0, 0.0).astype(jnp.float32)
        zero = jnp.zeros((), jnp.float32)
        rw = jnp.sum(jnp.where(hit0, w0_ref[...], zero)
                     + jnp.where(hit1, w1_ref[...], zero),
                     axis=1, keepdims=True)
        x = lax.dot_general(
            onehot, hid_ref[...], (((1,), (0,)), ((), ())),
            preferred_element_type=jnp.float32,
        )
        z = lax.dot_general(
            x, gu_v[slot], (((1,), (1,)), ((), ())),
            preferred_element_type=jnp.float32,
        )
        gate = z[:, :_F]
        up = z[:, _F:]
        act = gate * jax.nn.sigmoid(gate) * up

        @pl.when(rf_ref[b] == 1)
        def _wait_dn():
            _dn_copy(dn_hbm, dn_v, sd, be_ref[b], slot).wait()

        y = lax.dot_general(
            act, dn_v[slot], (((1,), (1,)), ((), ())),
            preferred_element_type=jnp.float32,
        )
        ys_ref[...] = y * rw


_tc_grid = pltpu.PrefetchScalarGridSpec(
    num_scalar_prefetch=5,
    grid=(_NB,),
    in_specs=[
        pl.BlockSpec((1, _T), lambda b, *_: (0, 0)),
        pl.BlockSpec((1, _T), lambda b, *_: (0, 0)),
        pl.BlockSpec((1, _T), lambda b, *_: (0, 0)),
        pl.BlockSpec((1, _T), lambda b, *_: (0, 0)),
        pl.BlockSpec((_T, _H), lambda b, *_: (0, 0)),
        pl.BlockSpec(memory_space=pl.ANY),
        pl.BlockSpec(memory_space=pl.ANY),
    ],
    out_specs=pl.BlockSpec((_B, _H), lambda b, *_: (b, 0)),
    scratch_shapes=[
        pltpu.VMEM((2, 2 * _F, _H), jnp.float32),
        pltpu.VMEM((2, _H, _F), jnp.float32),
        pltpu.SemaphoreType.DMA((2,)),
        pltpu.SemaphoreType.DMA((2,)),
    ],
)

_tc_moe = pl.pallas_call(
    _tc_body,
    grid_spec=_tc_grid,
    out_shape=jax.ShapeDtypeStruct((_NPAD, _H), jnp.float32),
)


def kernel(hidden_states, topk_weights, topk_ids, gate_up_weights, down_weights):
    ids = topk_ids.astype(jnp.int32)
    flat_e = ids.reshape(-1)
    onehot = (flat_e[:, None] == jnp.arange(_E, dtype=jnp.int32)[None, :]).astype(jnp.int32)
    prefix = jnp.cumsum(onehot, axis=0)
    counts = prefix[-1]
    rank = jnp.sum(onehot * prefix, axis=1) - 1
    padded = ((counts + _B - 1) // _B) * _B
    ends = jnp.cumsum(padded)
    seg_start = ends - padded
    dest = jnp.sum(onehot * seg_start[None, :], axis=1) + rank
    inv = dest.reshape(_T, _K)
    total = ends[-1]
    blk_start = jnp.arange(_NB, dtype=jnp.int32) * _B
    be = jnp.sum((blk_start[:, None] >= ends[None, :]).astype(jnp.int32),
                 axis=1)
    last_e = jnp.max(jnp.where(counts > 0, jnp.arange(_E, dtype=jnp.int32), 0))
    be = jnp.where(blk_start < total, jnp.minimum(be, _E - 1), last_e)

    rf = jnp.concatenate([jnp.ones((1,), jnp.int32),
                          (be[1:] != be[:-1]).astype(jnp.int32)])
    run_id = jnp.cumsum(rf) - 1
    slot_arr = (run_id % 2).astype(jnp.int32)
    cand = jnp.where(rf == 1, jnp.arange(_NB, dtype=jnp.int32), _NB)
    sufmin = lax.cummin(cand[::-1])[::-1]
    nxt = jnp.concatenate([sufmin[1:], jnp.full((1,), _NB, jnp.int32)])
    pf = jnp.where((rf == 1) & (nxt < _NB),
                   be[jnp.minimum(nxt, _NB - 1)], -1).astype(jnp.int32)
    real = (blk_start < total).astype(jnp.int32)

    i0 = inv[:, 0].astype(jnp.int32)
    i1 = inv[:, 1].astype(jnp.int32)
    ys = _tc_moe(be, rf, slot_arr, pf, real,
                 i0.reshape(1, _T), i1.reshape(1, _T),
                 topk_weights[:, 0].reshape(1, _T),
                 topk_weights[:, 1].reshape(1, _T),
                 hidden_states, gate_up_weights, down_weights)
    out = _sc_combine(ys, i0, i1)
    return out
